# baseline (speedup 1.0000x reference)
import functools

import jax
import jax.numpy as jnp
from jax.experimental import pallas as pl
from jax.experimental.pallas import tpu as pltpu

_B, _C, _H, _W = 64, 768, 24, 24
_HW = _H * _W
_HIDDEN = 192
_NE = 8


def _router_kernel(x_ref, w1_ref, b1_ref, w2_ref, b2_ref,
                   idx_ref, wgt_ref, pooled_ref):
    b = pl.program_id(0)
    row = jnp.sum(x_ref[0], axis=-1) * (1.0 / _HW)
    pooled_ref[pl.ds(b, 1), :] = row[None, :]

    @pl.when(b == _B - 1)
    def _finalize():
        pooled = pooled_ref[:, :]
        h = jnp.dot(pooled, w1_ref[:, :],
                    preferred_element_type=jnp.float32) + b1_ref[0]
        h = 0.5 * h * (1.0 + jax.lax.erf(h * (2.0 ** -0.5)))
        logits = jnp.dot(h, w2_ref[:, :],
                         preferred_element_type=jnp.float32) + b2_ref[0]

        eidx = jax.lax.broadcasted_iota(jnp.int32, (_B, _NE), 1)
        m1 = jnp.max(logits, axis=-1, keepdims=True)
        i1 = jnp.min(jnp.where(logits == m1, eidx, _NE), axis=-1, keepdims=True)
        masked = jnp.where(eidx == i1, -jnp.inf, logits)
        m2 = jnp.max(masked, axis=-1, keepdims=True)
        i2 = jnp.min(jnp.where(masked == m2, eidx, _NE), axis=-1, keepdims=True)

        e2 = jnp.exp(m2 - m1)
        denom = 1.0 + e2
        idx_ref[:, :] = jnp.concatenate([i1, i2], axis=1)
        wgt_ref[:, :] = jnp.concatenate([1.0 / denom, e2 / denom], axis=1)


@functools.partial(jax.jit, static_argnames=())
def kernel(x, W1, b1, W2, b2):
    xr = x.reshape(_B, _C, _HW)
    idx, wgt = pl.pallas_call(
        _router_kernel,
        grid=(_B,),
        in_specs=[
            pl.BlockSpec((1, _C, _HW), lambda b: (b, 0, 0)),
            pl.BlockSpec((_C, _HIDDEN), lambda b: (0, 0)),
            pl.BlockSpec((1, _HIDDEN), lambda b: (0, 0)),
            pl.BlockSpec((_HIDDEN, _NE), lambda b: (0, 0)),
            pl.BlockSpec((1, _NE), lambda b: (0, 0)),
        ],
        out_specs=[
            pl.BlockSpec((_B, 2), lambda b: (0, 0)),
            pl.BlockSpec((_B, 2), lambda b: (0, 0)),
        ],
        out_shape=[
            jax.ShapeDtypeStruct((_B, 2), jnp.int32),
            jax.ShapeDtypeStruct((_B, 2), jnp.float32),
        ],
        scratch_shapes=[pltpu.VMEM((_B, _C), jnp.float32)],
    )(xr, W1, b1.reshape(1, _HIDDEN), W2, b2.reshape(1, _NE))
    return idx, wgt


# speedup vs baseline: 1.0113x; 1.0113x over previous
import functools

import jax
import jax.numpy as jnp
from jax.experimental import pallas as pl
from jax.experimental.pallas import tpu as pltpu

_B, _C, _H, _W = 64, 768, 24, 24
_HW = _H * _W
_HIDDEN = 192
_NE = 8


def _router_kernel(x_ref, w1_ref, b1_ref, w2_ref, b2_ref,
                   idx_ref, wgt_ref, pooled_ref):
    b = pl.program_id(0)
    col = jnp.sum(x_ref[0], axis=-1, keepdims=True) * (1.0 / _HW)
    pooled_ref[pl.ds(b, 1)] = col[None]

    @pl.when(b == _B - 1)
    def _finalize():
        pooled = pooled_ref[:, :, 0]
        h = jnp.dot(pooled, w1_ref[:, :],
                    preferred_element_type=jnp.float32) + b1_ref[0]
        h = 0.5 * h * (1.0 + jax.lax.erf(h * (2.0 ** -0.5)))
        logits = jnp.dot(h, w2_ref[:, :],
                         preferred_element_type=jnp.float32) + b2_ref[0]

        eidx = jax.lax.broadcasted_iota(jnp.int32, (_B, _NE), 1)
        m1 = jnp.max(logits, axis=-1, keepdims=True)
        i1 = jnp.min(jnp.where(logits == m1, eidx, _NE), axis=-1, keepdims=True)
        masked = jnp.where(eidx == i1, -jnp.inf, logits)
        m2 = jnp.max(masked, axis=-1, keepdims=True)
        i2 = jnp.min(jnp.where(masked == m2, eidx, _NE), axis=-1, keepdims=True)

        e2 = jnp.exp(m2 - m1)
        denom = 1.0 + e2
        idx_ref[:, :] = jnp.concatenate([i1, i2], axis=1)
        wgt_ref[:, :] = jnp.concatenate([1.0 / denom, e2 / denom], axis=1)


@functools.partial(jax.jit, static_argnames=())
def kernel(x, W1, b1, W2, b2):
    xr = x.reshape(_B, _C, _HW)
    idx, wgt = pl.pallas_call(
        _router_kernel,
        grid=(_B,),
        in_specs=[
            pl.BlockSpec((1, _C, _HW), lambda b: (b, 0, 0)),
            pl.BlockSpec((_C, _HIDDEN), lambda b: (0, 0)),
            pl.BlockSpec((1, _HIDDEN), lambda b: (0, 0)),
            pl.BlockSpec((_HIDDEN, _NE), lambda b: (0, 0)),
            pl.BlockSpec((1, _NE), lambda b: (0, 0)),
        ],
        out_specs=[
            pl.BlockSpec((_B, 2), lambda b: (0, 0)),
            pl.BlockSpec((_B, 2), lambda b: (0, 0)),
        ],
        out_shape=[
            jax.ShapeDtypeStruct((_B, 2), jnp.int32),
            jax.ShapeDtypeStruct((_B, 2), jnp.float32),
        ],
        scratch_shapes=[pltpu.VMEM((_B, _C, 1), jnp.float32)],
    )(xr, W1, b1.reshape(1, _HIDDEN), W2, b2.reshape(1, _NE))
    return idx, wgt


# speedup vs baseline: 2.9059x; 2.8733x over previous
import functools

import jax
import jax.numpy as jnp
from jax.experimental import pallas as pl
from jax.experimental.pallas import tpu as pltpu

_B, _C, _H, _W = 64, 768, 24, 24
_HW = _H * _W
_HIDDEN = 192
_NE = 8


def _router_kernel(x_ref, w1_ref, b1_ref, w2_ref, b2_ref,
                   idx_ref, wgt_ref, pooled_ref):
    b = pl.program_id(0)
    pooled_ref[pl.ds(b, 1), :] = (
        jnp.sum(x_ref[0], axis=0, keepdims=True) * (1.0 / _HW))

    @pl.when(b == _B - 1)
    def _finalize():
        pooled = pooled_ref[:, :]
        h = jnp.dot(pooled, w1_ref[:, :],
                    preferred_element_type=jnp.float32) + b1_ref[0]
        h = 0.5 * h * (1.0 + jax.lax.erf(h * (2.0 ** -0.5)))
        logits = jnp.dot(h, w2_ref[:, :],
                         preferred_element_type=jnp.float32) + b2_ref[0]

        eidx = jax.lax.broadcasted_iota(jnp.int32, (_B, _NE), 1)
        m1 = jnp.max(logits, axis=-1, keepdims=True)
        i1 = jnp.min(jnp.where(logits == m1, eidx, _NE), axis=-1, keepdims=True)
        masked = jnp.where(eidx == i1, -jnp.inf, logits)
        m2 = jnp.max(masked, axis=-1, keepdims=True)
        i2 = jnp.min(jnp.where(masked == m2, eidx, _NE), axis=-1, keepdims=True)

        e2 = jnp.exp(m2 - m1)
        denom = 1.0 + e2
        idx_ref[:, :] = jnp.concatenate([i1, i2], axis=1)
        wgt_ref[:, :] = jnp.concatenate([1.0 / denom, e2 / denom], axis=1)


@functools.partial(jax.jit, static_argnames=())
def kernel(x, W1, b1, W2, b2):
    xt = jnp.transpose(x, (0, 2, 3, 1)).reshape(_B, _HW, _C)
    idx, wgt = pl.pallas_call(
        _router_kernel,
        grid=(_B,),
        in_specs=[
            pl.BlockSpec((1, _HW, _C), lambda b: (b, 0, 0)),
            pl.BlockSpec((_C, _HIDDEN), lambda b: (0, 0)),
            pl.BlockSpec((1, _HIDDEN), lambda b: (0, 0)),
            pl.BlockSpec((_HIDDEN, _NE), lambda b: (0, 0)),
            pl.BlockSpec((1, _NE), lambda b: (0, 0)),
        ],
        out_specs=[
            pl.BlockSpec((_B, 2), lambda b: (0, 0)),
            pl.BlockSpec((_B, 2), lambda b: (0, 0)),
        ],
        out_shape=[
            jax.ShapeDtypeStruct((_B, 2), jnp.int32),
            jax.ShapeDtypeStruct((_B, 2), jnp.float32),
        ],
        scratch_shapes=[pltpu.VMEM((_B, _C), jnp.float32)],
    )(xt, W1, b1.reshape(1, _HIDDEN), W2, b2.reshape(1, _NE))
    return idx, wgt


# speedup vs baseline: 4.1622x; 1.4324x over previous
import functools

import jax
import jax.numpy as jnp
from jax.experimental import pallas as pl
from jax.experimental.pallas import tpu as pltpu

_B, _C, _H, _W = 64, 768, 24, 24
_HW = _H * _W
_HIDDEN = 192
_NE = 8
_ROWS = 8


def _router_kernel(x_ref, w1_ref, b1_ref, w2_ref, b2_ref,
                   idx_ref, wgt_ref, pooled_ref):
    b = pl.program_id(0)
    pooled_ref[pl.ds(b * _ROWS, _ROWS), :] = (
        jnp.sum(x_ref[:, :, :], axis=1) * (1.0 / _HW))

    @pl.when(b == _B // _ROWS - 1)
    def _finalize():
        pooled = pooled_ref[:, :]
        h = jnp.dot(pooled, w1_ref[:, :],
                    preferred_element_type=jnp.float32) + b1_ref[0]
        h = 0.5 * h * (1.0 + jax.lax.erf(h * (2.0 ** -0.5)))
        logits = jnp.dot(h, w2_ref[:, :],
                         preferred_element_type=jnp.float32) + b2_ref[0]

        eidx = jax.lax.broadcasted_iota(jnp.int32, (_B, _NE), 1)
        m1 = jnp.max(logits, axis=-1, keepdims=True)
        i1 = jnp.min(jnp.where(logits == m1, eidx, _NE), axis=-1, keepdims=True)
        masked = jnp.where(eidx == i1, -jnp.inf, logits)
        m2 = jnp.max(masked, axis=-1, keepdims=True)
        i2 = jnp.min(jnp.where(masked == m2, eidx, _NE), axis=-1, keepdims=True)

        e2 = jnp.exp(m2 - m1)
        denom = 1.0 + e2
        idx_ref[:, :] = jnp.concatenate([i1, i2], axis=1)
        wgt_ref[:, :] = jnp.concatenate([1.0 / denom, e2 / denom], axis=1)


@functools.partial(jax.jit, static_argnames=())
def kernel(x, W1, b1, W2, b2):
    xt = jnp.transpose(x, (0, 2, 3, 1)).reshape(_B, _HW, _C)
    idx, wgt = pl.pallas_call(
        _router_kernel,
        grid=(_B // _ROWS,),
        in_specs=[
            pl.BlockSpec((_ROWS, _HW, _C), lambda b: (b, 0, 0)),
            pl.BlockSpec((_C, _HIDDEN), lambda b: (0, 0)),
            pl.BlockSpec((1, _HIDDEN), lambda b: (0, 0)),
            pl.BlockSpec((_HIDDEN, _NE), lambda b: (0, 0)),
            pl.BlockSpec((1, _NE), lambda b: (0, 0)),
        ],
        out_specs=[
            pl.BlockSpec((_B, 2), lambda b: (0, 0)),
            pl.BlockSpec((_B, 2), lambda b: (0, 0)),
        ],
        out_shape=[
            jax.ShapeDtypeStruct((_B, 2), jnp.int32),
            jax.ShapeDtypeStruct((_B, 2), jnp.float32),
        ],
        scratch_shapes=[pltpu.VMEM((_B, _C), jnp.float32)],
    )(xt, W1, b1.reshape(1, _HIDDEN), W2, b2.reshape(1, _NE))
    return idx, wgt
